# baseline (speedup 1.0000x reference)
import functools

import jax
import jax.numpy as jnp
from jax.experimental import pallas as pl

B = 64
C = 768
H = 14
W = 14
FREQ = 256
E = 64
K = 8


def _routing_body(x_ref, freq_ref, wg_ref, wf_ref, gates_ref, idx_ref, val_ref):
    xb = x_ref[0].reshape(C, H * W)
    pooled = jnp.sum(xb, axis=1, keepdims=True) * (1.0 / (H * W))

    logits = jax.lax.dot_general(
        pooled, wg_ref[...],
        dimension_numbers=(((0,), (1,)), ((), ())),
        preferred_element_type=jnp.float32,
    )
    logits += jax.lax.dot_general(
        freq_ref[0], wf_ref[...],
        dimension_numbers=(((1,), (1,)), ((), ())),
        preferred_element_type=jnp.float32,
    )

    m = jnp.max(logits, axis=-1, keepdims=True)
    ex = jnp.exp(logits - m)
    scores = ex / jnp.sum(ex, axis=-1, keepdims=True)

    iota = jax.lax.broadcasted_iota(jnp.int32, (1, E), 1)
    active = jnp.ones((1, E), dtype=jnp.bool_)
    gates = jnp.zeros((1, E), dtype=jnp.float32)
    idxs = []
    vals = []
    for _ in range(K):
        masked = jnp.where(active, scores, -jnp.inf)
        v = jnp.max(masked, axis=-1, keepdims=True)
        cand = jnp.where(masked == v, iota, E)
        i = jnp.min(cand, axis=-1, keepdims=True)
        gates = jnp.where(iota == i, v, gates)
        active = active & (iota != i)
        idxs.append(i)
        vals.append(v)

    gates_ref[0] = gates
    idx_ref[0] = jnp.concatenate(idxs, axis=-1)
    val_ref[0] = jnp.concatenate(vals, axis=-1)


@jax.jit
def kernel(x, freq_emb, W_gate, W_freq):
    gates3, idx3, val3 = pl.pallas_call(
        _routing_body,
        grid=(B,),
        in_specs=[
            pl.BlockSpec((1, C, H, W), lambda b: (b, 0, 0, 0)),
            pl.BlockSpec((1, 1, FREQ), lambda b: (b, 0, 0)),
            pl.BlockSpec((E, C), lambda b: (0, 0)),
            pl.BlockSpec((E, FREQ), lambda b: (0, 0)),
        ],
        out_specs=[
            pl.BlockSpec((1, 1, E), lambda b: (b, 0, 0)),
            pl.BlockSpec((1, 1, K), lambda b: (b, 0, 0)),
            pl.BlockSpec((1, 1, K), lambda b: (b, 0, 0)),
        ],
        out_shape=[
            jax.ShapeDtypeStruct((B, 1, E), jnp.float32),
            jax.ShapeDtypeStruct((B, 1, K), jnp.int32),
            jax.ShapeDtypeStruct((B, 1, K), jnp.float32),
        ],
    )(x, freq_emb.reshape(B, 1, FREQ), W_gate, W_freq)
    return gates3.reshape(B, E), idx3.reshape(B, K), val3.reshape(B, K)


# speedup vs baseline: 5.7289x; 5.7289x over previous
import jax
import jax.numpy as jnp
from jax.experimental import pallas as pl

B = 64
C = 768
H = 14
W = 14
HW = H * W
FREQ = 256
E = 64
K = 8
BB = 8


def _routing_body(x_ref, freq_ref, wg_ref, wf_ref, gates_ref, idx_ref, val_ref):
    pooled = jnp.sum(x_ref[...], axis=2) * (1.0 / HW)

    logits = jax.lax.dot_general(
        pooled, wg_ref[...],
        dimension_numbers=(((1,), (1,)), ((), ())),
        preferred_element_type=jnp.float32,
    )
    logits += jax.lax.dot_general(
        freq_ref[...], wf_ref[...],
        dimension_numbers=(((1,), (1,)), ((), ())),
        preferred_element_type=jnp.float32,
    )

    m = jnp.max(logits, axis=-1, keepdims=True)
    ex = jnp.exp(logits - m)
    scores = ex / jnp.sum(ex, axis=-1, keepdims=True)

    iota = jax.lax.broadcasted_iota(jnp.int32, (BB, E), 1)
    active = jnp.ones((BB, E), dtype=jnp.bool_)
    gates = jnp.zeros((BB, E), dtype=jnp.float32)
    idxs = []
    vals = []
    for _ in range(K):
        masked = jnp.where(active, scores, -jnp.inf)
        v = jnp.max(masked, axis=-1, keepdims=True)
        cand = jnp.where(masked == v, iota, E)
        i = jnp.min(cand, axis=-1, keepdims=True)
        gates = jnp.where(iota == i, v, gates)
        active = active & (iota != i)
        idxs.append(i)
        vals.append(v)

    gates_ref[...] = gates
    idx_ref[...] = jnp.concatenate(idxs, axis=-1)
    val_ref[...] = jnp.concatenate(vals, axis=-1)


@jax.jit
def kernel(x, freq_emb, W_gate, W_freq):
    gates, idx, val = pl.pallas_call(
        _routing_body,
        grid=(B // BB,),
        in_specs=[
            pl.BlockSpec((BB, C, HW), lambda b: (b, 0, 0)),
            pl.BlockSpec((BB, FREQ), lambda b: (b, 0)),
            pl.BlockSpec((E, C), lambda b: (0, 0)),
            pl.BlockSpec((E, FREQ), lambda b: (0, 0)),
        ],
        out_specs=[
            pl.BlockSpec((BB, E), lambda b: (b, 0)),
            pl.BlockSpec((BB, K), lambda b: (b, 0)),
            pl.BlockSpec((BB, K), lambda b: (b, 0)),
        ],
        out_shape=[
            jax.ShapeDtypeStruct((B, E), jnp.float32),
            jax.ShapeDtypeStruct((B, K), jnp.int32),
            jax.ShapeDtypeStruct((B, K), jnp.float32),
        ],
    )(x.reshape(B, C, HW), freq_emb, W_gate, W_freq)
    return gates, idx, val
